# baseline (speedup 1.0000x reference)
import functools

import jax
import jax.numpy as jnp
from jax import lax
from jax.experimental import pallas as pl
from jax.experimental.pallas import tpu as pltpu
from jax.experimental.pallas import tpu_sc as plsc

N = 10000
E = 160000
D = 256
H = 256

NC = 2
NS = 16
NW = NC * NS
LANES = 16
CH = 128
HD = D // 2

N_PAD = 10240
SL = N_PAD // NS

NCH_D = -(-E // (NW * CH))
EW_D = NCH_D * CH
CHM = 40
EW_M = E // NS
NCH_M = EW_M // CHM
SUP = 16
NSUP = 16
NCH_PAD = NSUP * SUP
NBUF = 4

ROWS2 = 1280
ROWS4 = 1280
ROWS6 = 3200

_mesh = plsc.VectorSubcoreMesh(core_axis_name="c", subcore_axis_name="s")


@functools.partial(
    pl.kernel,
    out_type=jax.ShapeDtypeStruct((NC, 2, N_PAD), jnp.float32),
    mesh=_mesh,
    scratch_types=[
        pltpu.VMEM((NCH_D, CH), jnp.int32),
        pltpu.VMEM((NCH_D, CH), jnp.int32),
        pltpu.VMEM((CH,), jnp.float32),
        pltpu.VMEM((SL,), jnp.float32),
        pltpu.VMEM_SHARED((N_PAD,), jnp.float32),
        pltpu.VMEM_SHARED((N_PAD,), jnp.float32),
    ],
)
def _deg_kernel(src_hbm, dst_hbm, out_hbm, src_v, dst_v, ones_v, zero_v,
                sh_in, sh_out):
    c = lax.axis_index("c")
    s = lax.axis_index("s")
    w = c * NS + s

    for g in range(CH // LANES):
        ones_v[pl.ds(g * LANES, LANES)] = jnp.full((LANES,), 1.0, jnp.float32)
    for g in range(SL // LANES):
        zero_v[pl.ds(g * LANES, LANES)] = jnp.zeros((LANES,), jnp.float32)

    pltpu.sync_copy(zero_v, sh_in.at[pl.ds(s * SL, SL)])
    pltpu.sync_copy(zero_v, sh_out.at[pl.ds(s * SL, SL)])
    plsc.subcore_barrier()

    pltpu.sync_copy(src_hbm.at[w], src_v)
    pltpu.sync_copy(dst_hbm.at[w], dst_v)

    nch = jnp.minimum(E - w * EW_D, EW_D) // CH

    def body(j, carry):
        pltpu.sync_copy(ones_v, sh_in.at[dst_v.at[j]], add=True)
        pltpu.sync_copy(ones_v, sh_out.at[src_v.at[j]], add=True)
        return carry

    lax.fori_loop(0, nch, body, 0)
    plsc.subcore_barrier()

    pltpu.sync_copy(sh_in.at[pl.ds(s * SL, SL)],
                    out_hbm.at[c, 0, pl.ds(s * SL, SL)])
    pltpu.sync_copy(sh_out.at[pl.ds(s * SL, SL)],
                    out_hbm.at[c, 1, pl.ds(s * SL, SL)])


@functools.partial(
    pl.kernel,
    out_type=jax.ShapeDtypeStruct((NC, N_PAD, HD), jnp.float32),
    mesh=_mesh,
    scratch_types=(
        [pltpu.VMEM((SUP, CHM), jnp.int32)] * 2 +
        [pltpu.VMEM((CHM, HD), jnp.float32)] * (2 * NBUF) +
        [pltpu.VMEM_SHARED((N_PAD, HD), jnp.float32)] +
        [pltpu.SemaphoreType.DMA] * (2 * NBUF)
    ),
)
def _scatter_kernel(srcoff_hbm, dst_hbm, node_hbm, edge_hbm, out_hbm,
                    src_v, dst_v, n0, n1, n2, n3, e0, e1, e2, e3, sh_seg,
                    i0, i1, i2, i3, q0, q1, q2, q3):
    c = lax.axis_index("c")
    s = lax.axis_index("s")
    nodes = (n0, n1, n2, n3)
    edges = (e0, e1, e2, e3)
    insem = (i0, i1, i2, i3)
    scsem = (q0, q1, q2, q3)
    edge0 = e0

    def zrow(r, carry):
        for g in range(HD // LANES):
            edge0[r, pl.ds(g * LANES, LANES)] = jnp.zeros((LANES,),
                                                          jnp.float32)
        return carry

    lax.fori_loop(0, CHM, zrow, 0)

    def zseg(i, carry):
        pltpu.sync_copy(edge0, sh_seg.at[pl.ds(s * SL + i * CHM, CHM)])
        return carry

    lax.fori_loop(0, SL // CHM, zseg, 0)
    plsc.subcore_barrier()

    def issue_in(jj, j, par):
        pltpu.async_copy(node_hbm.at[src_v.at[jj]], nodes[par], insem[par])
        base = jnp.minimum(s * EW_M + j * CHM, E - CHM)
        pltpu.async_copy(edge_hbm.at[pl.ds(base, CHM), pl.ds(c * HD, HD)],
                         edges[par], insem[par])

    def wait_in(jj, j, par):
        pltpu.make_async_copy(node_hbm.at[src_v.at[jj]], nodes[par],
                              insem[par]).wait()
        base = jnp.minimum(s * EW_M + j * CHM, E - CHM)
        pltpu.make_async_copy(
            edge_hbm.at[pl.ds(base, CHM), pl.ds(c * HD, HD)],
            edges[par], insem[par]).wait()

    def issue_scat(jj, par):
        pltpu.async_copy(nodes[par], sh_seg.at[dst_v.at[jj]], scsem[par],
                         add=True)

    def wait_scat(jj, par):
        pltpu.make_async_copy(nodes[par], sh_seg.at[dst_v.at[jj]],
                              scsem[par]).wait()

    def compute(par):
        nv, ev = nodes[par], edges[par]

        def rowmul(e, ccc):
            for g in range(HD // LANES):
                sl = pl.ds(g * LANES, LANES)
                nv[e, sl] = nv[e, sl] * ev[e, sl]
            return ccc

        lax.fori_loop(0, CHM, rowmul, 0)

    for s8 in range(NSUP):
        pltpu.sync_copy(srcoff_hbm.at[c, s, pl.ds(s8 * SUP, SUP)], src_v)
        pltpu.sync_copy(dst_hbm.at[s, pl.ds(s8 * SUP, SUP)], dst_v)
        if s8 > 0:
            wait_scat(SUP - 4, 0)
            wait_scat(SUP - 3, 1)
        issue_in(0, s8 * SUP + 0, 0)
        issue_in(1, s8 * SUP + 1, 1)

        def grp(g4, carry):
            for k in range(NBUF):
                jl = g4 * NBUF + k
                wait_in(jl, s8 * SUP + jl, k)
                compute(k)
                issue_scat(jl, k)
                tb = (k + 2) % NBUF

                @pl.when(jl <= SUP - 3)
                def _(jl=jl, tb=tb):
                    if s8 > 0:
                        wait_scat(jl, tb)
                        issue_in(jl + 2, s8 * SUP + jl + 2, tb)
                    else:
                        @pl.when(jl >= 2)
                        def _():
                            wait_scat(jl, tb)

                        issue_in(jl + 2, s8 * SUP + jl + 2, tb)

            return carry

        lax.fori_loop(0, SUP // NBUF, grp, 0)

    wait_scat(SUP - 4, 0)
    wait_scat(SUP - 3, 1)
    wait_scat(SUP - 2, 2)
    wait_scat(SUP - 1, 3)
    plsc.subcore_barrier()

    pltpu.sync_copy(sh_seg.at[pl.ds(s * SL, SL)],
                    out_hbm.at[c, pl.ds(s * SL, SL)])


def _scale_body(node_ref, deg_ref, out_ref):
    s_out = lax.rsqrt(deg_ref[1, :] + deg_ref[3, :] + 1.0)
    out_ref[...] = node_ref[...] * s_out[None, :, None]


def _fused_body(seg_ref, node_ref, deg_ref, wlo_ref, whi_ref, lw_ref, lr_ref,
                b_ref, pre_ref, sums_ref):
    g = pl.program_id(0)
    agg = (jnp.dot(seg_ref[0], wlo_ref[...],
                   preferred_element_type=jnp.float32) +
           jnp.dot(seg_ref[1], whi_ref[...],
                   preferred_element_type=jnp.float32))
    s_in = lax.rsqrt(deg_ref[0, :] + deg_ref[2, :] + 1.0)
    loop_msg = jnp.dot(node_ref[...] * lr_ref[...], lw_ref[...],
                       preferred_element_type=jnp.float32)
    pre = (agg * s_in[:, None] + loop_msg) * 0.3333333 + b_ref[...]
    pre_ref[...] = pre

    rid = g * ROWS4 + lax.broadcasted_iota(jnp.int32, (ROWS4, 1), 0)
    pm = jnp.where(rid < N, pre, 0.0)
    colsum = jnp.sum(pm, axis=0)
    colsq = jnp.sum(pm * pm, axis=0)

    @pl.when(g == 0)
    def _():
        sums_ref[0, :] = colsum
        sums_ref[1, :] = colsq

    @pl.when(g > 0)
    def _():
        sums_ref[0, :] = sums_ref[0, :] + colsum
        sums_ref[1, :] = sums_ref[1, :] + colsq


def _bn_body(pre_ref, sums_ref, gam_ref, bet_ref, out_ref):
    inv_n = jnp.float32(1.0 / N)
    mean = sums_ref[0, :] * inv_n
    var = sums_ref[1, :] * inv_n - mean * mean
    inv = lax.rsqrt(var + 1e-5)
    y = (pre_ref[...] - mean[None, :]) * (inv * gam_ref[0, :])[None, :] \
        + bet_ref[...]
    out_ref[...] = jnp.maximum(y, 0.0)


def _mm_body(x_ref, w_ref, o_ref):
    o_ref[...] = jnp.dot(x_ref[...], w_ref[...],
                         preferred_element_type=jnp.float32)


def kernel(node_feat, edge_index, edge_feat, in_weight, rel_weight,
           loop_weight, loop_rel, bias, bn_gamma, bn_beta):
    src = edge_index[0].astype(jnp.int32)
    dst = edge_index[1].astype(jnp.int32)

    pad_d = NW * EW_D - E
    src_d = jnp.pad(src, (0, pad_d), constant_values=N).reshape(NW, NCH_D, CH)
    dst_d = jnp.pad(dst, (0, pad_d), constant_values=N).reshape(NW, NCH_D, CH)

    row_pad = ((0, 0), (0, NCH_PAD - NCH_M), (0, 0))
    src_m = jnp.pad(src.reshape(NS, NCH_M, CHM), row_pad)
    src_off = jnp.stack([src_m, src_m + N])
    dst_m = jnp.pad(dst.reshape(NS, NCH_M, CHM), row_pad, constant_values=N)

    node_cat = jnp.stack([node_feat[:, :HD], node_feat[:, HD:]])

    deg = _deg_kernel(src_d, dst_d).reshape(4, N_PAD)

    node_scaled = pl.pallas_call(
        _scale_body,
        grid=(-(-N // ROWS2),),
        in_specs=[
            pl.BlockSpec((NC, ROWS2, HD), lambda g: (0, g, 0)),
            pl.BlockSpec((4, ROWS2), lambda g: (0, g)),
        ],
        out_specs=pl.BlockSpec((NC, ROWS2, HD), lambda g: (0, g, 0)),
        out_shape=jax.ShapeDtypeStruct((NC, N, HD), jnp.float32),
    )(node_cat, deg).reshape(NC * N, HD)

    seg = _scatter_kernel(src_off, dst_m, node_scaled, edge_feat)

    edge_out = pl.pallas_call(
        _mm_body,
        grid=(E // ROWS6,),
        in_specs=[
            pl.BlockSpec((ROWS6, D), lambda g: (g, 0)),
            pl.BlockSpec((D, H), lambda g: (0, 0)),
        ],
        out_specs=pl.BlockSpec((ROWS6, H), lambda g: (g, 0)),
        out_shape=jax.ShapeDtypeStruct((E, H), jnp.float32),
    )(edge_feat, rel_weight)

    nsteps = -(-N // ROWS4)
    pre, sums = pl.pallas_call(
        _fused_body,
        grid=(nsteps,),
        in_specs=[
            pl.BlockSpec((NC, ROWS4, HD), lambda g: (0, g, 0)),
            pl.BlockSpec((ROWS4, D), lambda g: (g, 0)),
            pl.BlockSpec((4, ROWS4), lambda g: (0, g)),
            pl.BlockSpec((HD, H), lambda g: (0, 0)),
            pl.BlockSpec((HD, H), lambda g: (0, 0)),
            pl.BlockSpec((D, H), lambda g: (0, 0)),
            pl.BlockSpec((1, D), lambda g: (0, 0)),
            pl.BlockSpec((1, H), lambda g: (0, 0)),
        ],
        out_specs=[
            pl.BlockSpec((ROWS4, H), lambda g: (g, 0)),
            pl.BlockSpec((2, H), lambda g: (0, 0)),
        ],
        out_shape=[
            jax.ShapeDtypeStruct((N, H), jnp.float32),
            jax.ShapeDtypeStruct((2, H), jnp.float32),
        ],
    )(seg, node_feat, deg, in_weight[:HD], in_weight[HD:], loop_weight,
      loop_rel, bias.reshape(1, H))

    out = pl.pallas_call(
        _bn_body,
        grid=(nsteps,),
        in_specs=[
            pl.BlockSpec((ROWS4, H), lambda g: (g, 0)),
            pl.BlockSpec((2, H), lambda g: (0, 0)),
            pl.BlockSpec((1, H), lambda g: (0, 0)),
            pl.BlockSpec((1, H), lambda g: (0, 0)),
        ],
        out_specs=pl.BlockSpec((ROWS4, H), lambda g: (g, 0)),
        out_shape=jax.ShapeDtypeStruct((N, H), jnp.float32),
    )(pre, sums, bn_gamma.reshape(1, H), bn_beta.reshape(1, H))

    return (out, edge_out)


# speedup vs baseline: 1.0488x; 1.0488x over previous
import functools

import jax
import jax.numpy as jnp
from jax import lax
from jax.experimental import pallas as pl
from jax.experimental.pallas import tpu as pltpu
from jax.experimental.pallas import tpu_sc as plsc

N = 10000
E = 160000
D = 256
H = 256

NC = 2
NS = 16
NW = NC * NS
LANES = 16
CH = 128
HD = D // 2

N_PAD = 10240
SL = N_PAD // NS

NCH_D = -(-E // (NW * CH))
EW_D = NCH_D * CH
CHM = 128
NCH_M = -(-E // (NS * CHM))
EW_M = NCH_M * CHM
SUP = 8
NCH_PAD = -(-NCH_M // SUP) * SUP

ROWS2 = 1280
ROWS4 = 1280
ROWS6 = 3200

_mesh = plsc.VectorSubcoreMesh(core_axis_name="c", subcore_axis_name="s")


@functools.partial(
    pl.kernel,
    out_type=jax.ShapeDtypeStruct((NC, 2, N_PAD), jnp.float32),
    mesh=_mesh,
    scratch_types=[
        pltpu.VMEM((NCH_D, CH), jnp.int32),
        pltpu.VMEM((NCH_D, CH), jnp.int32),
        pltpu.VMEM((CH,), jnp.float32),
        pltpu.VMEM((SL,), jnp.float32),
        pltpu.VMEM_SHARED((N_PAD,), jnp.float32),
        pltpu.VMEM_SHARED((N_PAD,), jnp.float32),
    ],
)
def _deg_kernel(src_hbm, dst_hbm, out_hbm, src_v, dst_v, ones_v, zero_v,
                sh_in, sh_out):
    c = lax.axis_index("c")
    s = lax.axis_index("s")
    w = c * NS + s

    for g in range(CH // LANES):
        ones_v[pl.ds(g * LANES, LANES)] = jnp.full((LANES,), 1.0, jnp.float32)
    for g in range(SL // LANES):
        zero_v[pl.ds(g * LANES, LANES)] = jnp.zeros((LANES,), jnp.float32)

    pltpu.sync_copy(zero_v, sh_in.at[pl.ds(s * SL, SL)])
    pltpu.sync_copy(zero_v, sh_out.at[pl.ds(s * SL, SL)])
    plsc.subcore_barrier()

    pltpu.sync_copy(src_hbm.at[w], src_v)
    pltpu.sync_copy(dst_hbm.at[w], dst_v)

    nch = jnp.minimum(E - w * EW_D, EW_D) // CH

    def body(j, carry):
        pltpu.sync_copy(ones_v, sh_in.at[dst_v.at[j]], add=True)
        pltpu.sync_copy(ones_v, sh_out.at[src_v.at[j]], add=True)
        return carry

    lax.fori_loop(0, nch, body, 0)
    plsc.subcore_barrier()

    pltpu.sync_copy(sh_in.at[pl.ds(s * SL, SL)],
                    out_hbm.at[c, 0, pl.ds(s * SL, SL)])
    pltpu.sync_copy(sh_out.at[pl.ds(s * SL, SL)],
                    out_hbm.at[c, 1, pl.ds(s * SL, SL)])


@functools.partial(
    pl.kernel,
    out_type=jax.ShapeDtypeStruct((NC, N_PAD, HD), jnp.float32),
    mesh=_mesh,
    scratch_types=[
        pltpu.VMEM((SUP, CHM), jnp.int32),
        pltpu.VMEM((SUP, CHM), jnp.int32),
        pltpu.VMEM((CHM, HD), jnp.float32),
        pltpu.VMEM((CHM, HD), jnp.float32),
        pltpu.VMEM_SHARED((N_PAD, HD), jnp.float32),
        pltpu.SemaphoreType.DMA,
    ],
)
def _scatter_kernel(srcoff_hbm, dst_hbm, node_hbm, edge_hbm, out_hbm,
                    src_v, dst_v, node_v, edge_v, sh_seg, sem):
    c = lax.axis_index("c")
    s = lax.axis_index("s")

    def zrow(r, carry):
        for g in range(HD // LANES):
            edge_v[r, pl.ds(g * LANES, LANES)] = jnp.zeros((LANES,),
                                                           jnp.float32)
        return carry

    lax.fori_loop(0, CHM, zrow, 0)

    def zseg(i, carry):
        pltpu.sync_copy(edge_v, sh_seg.at[pl.ds(s * SL + i * CHM, CHM)])
        return carry

    lax.fori_loop(0, SL // CHM, zseg, 0)
    plsc.subcore_barrier()

    nch = jnp.minimum(E - s * EW_M, EW_M) // CHM

    def super_chunk(sc_i, carry):
        pltpu.sync_copy(srcoff_hbm.at[c, s, pl.ds(sc_i * SUP, SUP)], src_v)
        pltpu.sync_copy(dst_hbm.at[s, pl.ds(sc_i * SUP, SUP)], dst_v)

        def chunk(jj, cc):
            j = sc_i * SUP + jj
            pltpu.async_copy(node_hbm.at[src_v.at[jj]], node_v, sem).wait()
            pltpu.sync_copy(
                edge_hbm.at[pl.ds(s * EW_M + j * CHM, CHM),
                            pl.ds(c * HD, HD)], edge_v)

            def rowmul(e, ccc):
                for g in range(HD // LANES):
                    sl = pl.ds(g * LANES, LANES)
                    node_v[e, sl] = node_v[e, sl] * edge_v[e, sl]
                return ccc

            lax.fori_loop(0, CHM, rowmul, 0)
            pltpu.sync_copy(node_v, sh_seg.at[dst_v.at[jj]], add=True)
            return cc

        lax.fori_loop(0, jnp.minimum(nch - sc_i * SUP, SUP), chunk, 0)
        return carry

    lax.fori_loop(0, (nch + SUP - 1) // SUP, super_chunk, 0)
    plsc.subcore_barrier()

    pltpu.sync_copy(sh_seg.at[pl.ds(s * SL, SL)],
                    out_hbm.at[c, pl.ds(s * SL, SL)])


def _scale_body(node_ref, deg_ref, out_ref):
    s_out = lax.rsqrt(deg_ref[1, :] + deg_ref[3, :] + 1.0)
    out_ref[...] = node_ref[...] * s_out[None, :, None]


def _fused_body(seg_ref, node_ref, deg_ref, wlo_ref, whi_ref, lw_ref, lr_ref,
                b_ref, pre_ref, sums_ref):
    g = pl.program_id(0)
    agg = (jnp.dot(seg_ref[0], wlo_ref[...],
                   preferred_element_type=jnp.float32) +
           jnp.dot(seg_ref[1], whi_ref[...],
                   preferred_element_type=jnp.float32))
    s_in = lax.rsqrt(deg_ref[0, :] + deg_ref[2, :] + 1.0)
    loop_msg = jnp.dot(node_ref[...] * lr_ref[...], lw_ref[...],
                       preferred_element_type=jnp.float32)
    pre = (agg * s_in[:, None] + loop_msg) * 0.3333333 + b_ref[...]
    pre_ref[...] = pre

    rid = g * ROWS4 + lax.broadcasted_iota(jnp.int32, (ROWS4, 1), 0)
    pm = jnp.where(rid < N, pre, 0.0)
    colsum = jnp.sum(pm, axis=0)
    colsq = jnp.sum(pm * pm, axis=0)

    @pl.when(g == 0)
    def _():
        sums_ref[0, :] = colsum
        sums_ref[1, :] = colsq

    @pl.when(g > 0)
    def _():
        sums_ref[0, :] = sums_ref[0, :] + colsum
        sums_ref[1, :] = sums_ref[1, :] + colsq


def _bn_body(pre_ref, sums_ref, gam_ref, bet_ref, out_ref):
    inv_n = jnp.float32(1.0 / N)
    mean = sums_ref[0, :] * inv_n
    var = sums_ref[1, :] * inv_n - mean * mean
    inv = lax.rsqrt(var + 1e-5)
    y = (pre_ref[...] - mean[None, :]) * (inv * gam_ref[0, :])[None, :] \
        + bet_ref[...]
    out_ref[...] = jnp.maximum(y, 0.0)


def _mm_body(x_ref, w_ref, o_ref):
    o_ref[...] = jnp.dot(x_ref[...], w_ref[...],
                         preferred_element_type=jnp.float32)


def kernel(node_feat, edge_index, edge_feat, in_weight, rel_weight,
           loop_weight, loop_rel, bias, bn_gamma, bn_beta):
    src = edge_index[0].astype(jnp.int32)
    dst = edge_index[1].astype(jnp.int32)

    pad_d = NW * EW_D - E
    src_d = jnp.pad(src, (0, pad_d), constant_values=N).reshape(NW, NCH_D, CH)
    dst_d = jnp.pad(dst, (0, pad_d), constant_values=N).reshape(NW, NCH_D, CH)

    row_pad = ((0, 0), (0, NCH_PAD - NCH_M), (0, 0))
    src_m = jnp.pad(jnp.pad(src, (0, NS * EW_M - E)).reshape(NS, NCH_M, CHM),
                    row_pad)
    src_off = jnp.stack([src_m, src_m + N])
    dst_m = jnp.pad(
        jnp.pad(dst, (0, NS * EW_M - E),
                constant_values=N).reshape(NS, NCH_M, CHM),
        row_pad, constant_values=N)

    node_cat = jnp.stack([node_feat[:, :HD], node_feat[:, HD:]])

    deg = _deg_kernel(src_d, dst_d).reshape(4, N_PAD)

    node_scaled = pl.pallas_call(
        _scale_body,
        grid=(-(-N // ROWS2),),
        in_specs=[
            pl.BlockSpec((NC, ROWS2, HD), lambda g: (0, g, 0)),
            pl.BlockSpec((4, ROWS2), lambda g: (0, g)),
        ],
        out_specs=pl.BlockSpec((NC, ROWS2, HD), lambda g: (0, g, 0)),
        out_shape=jax.ShapeDtypeStruct((NC, N, HD), jnp.float32),
    )(node_cat, deg).reshape(NC * N, HD)

    seg = _scatter_kernel(src_off, dst_m, node_scaled, edge_feat)

    edge_out = pl.pallas_call(
        _mm_body,
        grid=(E // ROWS6,),
        in_specs=[
            pl.BlockSpec((ROWS6, D), lambda g: (g, 0)),
            pl.BlockSpec((D, H), lambda g: (0, 0)),
        ],
        out_specs=pl.BlockSpec((ROWS6, H), lambda g: (g, 0)),
        out_shape=jax.ShapeDtypeStruct((E, H), jnp.float32),
    )(edge_feat, rel_weight)

    nsteps = -(-N // ROWS4)
    pre, sums = pl.pallas_call(
        _fused_body,
        grid=(nsteps,),
        in_specs=[
            pl.BlockSpec((NC, ROWS4, HD), lambda g: (0, g, 0)),
            pl.BlockSpec((ROWS4, D), lambda g: (g, 0)),
            pl.BlockSpec((4, ROWS4), lambda g: (0, g)),
            pl.BlockSpec((HD, H), lambda g: (0, 0)),
            pl.BlockSpec((HD, H), lambda g: (0, 0)),
            pl.BlockSpec((D, H), lambda g: (0, 0)),
            pl.BlockSpec((1, D), lambda g: (0, 0)),
            pl.BlockSpec((1, H), lambda g: (0, 0)),
        ],
        out_specs=[
            pl.BlockSpec((ROWS4, H), lambda g: (g, 0)),
            pl.BlockSpec((2, H), lambda g: (0, 0)),
        ],
        out_shape=[
            jax.ShapeDtypeStruct((N, H), jnp.float32),
            jax.ShapeDtypeStruct((2, H), jnp.float32),
        ],
    )(seg, node_feat, deg, in_weight[:HD], in_weight[HD:], loop_weight,
      loop_rel, bias.reshape(1, H))

    out = pl.pallas_call(
        _bn_body,
        grid=(nsteps,),
        in_specs=[
            pl.BlockSpec((ROWS4, H), lambda g: (g, 0)),
            pl.BlockSpec((2, H), lambda g: (0, 0)),
            pl.BlockSpec((1, H), lambda g: (0, 0)),
            pl.BlockSpec((1, H), lambda g: (0, 0)),
        ],
        out_specs=pl.BlockSpec((ROWS4, H), lambda g: (g, 0)),
        out_shape=jax.ShapeDtypeStruct((N, H), jnp.float32),
    )(pre, sums, bn_gamma.reshape(1, H), bn_beta.reshape(1, H))

    return (out, edge_out)
